# baseline (device time: 140217 ns/iter reference)
import os

import jax
import jax.numpy as jnp
from jax import lax
from jax.experimental import pallas as pl
from jax.experimental.pallas import tpu as pltpu

N_DEV = 4

_PROBE = os.environ.get("PROBE_GEMM", "")


def _kernel_probe(x, w_mat, scale_x, scale_w):
    m_per, k = x.shape
    _, n_per = w_mat.shape
    xq = x.astype(jnp.float8_e4m3fn)
    wq = w_mat.astype(jnp.float8_e5m2)

    def body(x_ref, w_ref, sx_ref, sw_ref, out_ref):
        scale = sx_ref[0] * sw_ref[0]
        for h in range(N_DEV):
            acc = lax.dot_general(
                x_ref[...], w_ref[...],
                (((1,), (0,)), ((), ())),
                preferred_element_type=jnp.float32,
            )
            y = acc * scale
            out_ref[pl.ds(h * m_per, m_per), :] = y * jax.nn.sigmoid(y)

    return pl.pallas_call(
        body,
        out_shape=jax.ShapeDtypeStruct((N_DEV * m_per, n_per), jnp.float32),
        in_specs=[
            pl.BlockSpec(memory_space=pltpu.VMEM),
            pl.BlockSpec(memory_space=pltpu.VMEM),
            pl.BlockSpec(memory_space=pltpu.SMEM),
            pl.BlockSpec(memory_space=pltpu.SMEM),
        ],
        out_specs=pl.BlockSpec(memory_space=pltpu.VMEM),
        compiler_params=pltpu.CompilerParams(
            vmem_limit_bytes=100 * 1024 * 1024,
        ),
    )(xq, wq, scale_x, scale_w)


def _kernel_real(x, w_mat, scale_x, scale_w):
    m_per, k = x.shape
    k2, n_per = w_mat.shape
    assert k2 == k
    half = m_per // 2
    KT = 16
    kt = k // KT

    xq_in = x.astype(jnp.float8_e4m3fn)

    def body(x_ref, w_ref, sx_ref, sw_ref, out_init_ref, out_ref,
             wstage, wq, bufL, bufR, bufO, send_sems, recv_sems, wsem):
        del out_init_ref
        my = lax.axis_index("i")
        left = lax.rem(my + (N_DEV - 1), N_DEV)
        right = lax.rem(my + 1, N_DEV)

        barrier_sem = pltpu.get_barrier_semaphore()
        for nbr in (left, right):
            pl.semaphore_signal(
                barrier_sem, inc=1,
                device_id=(nbr,), device_id_type=pl.DeviceIdType.MESH,
            )
        pl.semaphore_wait(barrier_sem, 2)

        scale = sx_ref[0] * sw_ref[0]

        def gemm_store(chunk, origin):
            acc = lax.dot_general(
                chunk, wq[...],
                (((1,), (0,)), ((), ())),
                preferred_element_type=jnp.float32,
            )
            y = acc * scale
            out_ref[pl.ds(origin * m_per, m_per), :] = y * jax.nn.sigmoid(y)

        def rcopy(src, dst, i, dev):
            return pltpu.make_async_remote_copy(
                src_ref=src, dst_ref=dst,
                send_sem=send_sems.at[i], recv_sem=recv_sems.at[i],
                device_id=(dev,), device_id_type=pl.DeviceIdType.MESH,
            )

        r1 = rcopy(x_ref, bufL, 0, right)
        r2 = rcopy(x_ref, bufR, 1, left)
        r1.start()
        r2.start()
        def wcopy(t, slot):
            return pltpu.make_async_copy(
                w_ref.at[pl.ds(t * kt, kt)], wstage.at[slot], wsem.at[slot],
            )
        wcopy(0, 0).start()
        for t in range(KT):
            if t + 1 < KT:
                wcopy(t + 1, (t + 1) % 2).start()
            wcopy(t, t % 2).wait()
            wq[pl.ds(t * kt, kt), :] = wstage[t % 2].astype(jnp.float8_e5m2)
        gemm_store(x_ref[...], my)
        r1.wait_recv()
        r3 = rcopy(bufL.at[pl.ds(0, half)], bufO.at[pl.ds(0, half)], 2, right)
        r3.start()
        r2.wait_recv()
        r4 = rcopy(bufR.at[pl.ds(half, half)], bufO.at[pl.ds(half, half)], 3, left)
        r4.start()
        gemm_store(bufL[...], lax.rem(my + (N_DEV - 1), N_DEV))
        gemm_store(bufR[...], lax.rem(my + 1, N_DEV))
        r3.wait_recv()
        r4.wait_recv()
        gemm_store(bufO[...], lax.rem(my + 2, N_DEV))
        for r in (r1, r2, r3, r4):
            r.wait_send()

    return pl.pallas_call(
        body,
        out_shape=jax.ShapeDtypeStruct((N_DEV * m_per, n_per), jnp.float32),
        in_specs=[
            pl.BlockSpec(memory_space=pltpu.VMEM),
            pl.BlockSpec(memory_space=pl.ANY),
            pl.BlockSpec(memory_space=pltpu.SMEM),
            pl.BlockSpec(memory_space=pltpu.SMEM),
            pl.BlockSpec(memory_space=pl.ANY),
        ],
        out_specs=pl.BlockSpec(memory_space=pltpu.VMEM),
        input_output_aliases={4: 0},
        scratch_shapes=[
            pltpu.VMEM((2, kt, n_per), jnp.float32),
            pltpu.VMEM((k, n_per), jnp.float8_e5m2),
            pltpu.VMEM((m_per, k), jnp.float8_e4m3fn),
            pltpu.VMEM((m_per, k), jnp.float8_e4m3fn),
            pltpu.VMEM((m_per, k), jnp.float8_e4m3fn),
            pltpu.SemaphoreType.DMA((4,)),
            pltpu.SemaphoreType.DMA((4,)),
            pltpu.SemaphoreType.DMA((2,)),
        ],
        compiler_params=pltpu.CompilerParams(
            collective_id=0,
            vmem_limit_bytes=63 * 1024 * 1024,
        ),
    )(xq_in, w_mat, scale_x, scale_w,
      jnp.zeros((N_DEV * m_per, n_per), jnp.float32))


kernel = _kernel_probe if _PROBE else _kernel_real


# device time: 121752 ns/iter; 1.1517x vs baseline; 1.1517x over previous
import os

import jax
import jax.numpy as jnp
from jax import lax
from jax.experimental import pallas as pl
from jax.experimental.pallas import tpu as pltpu

N_DEV = 4

_PROBE = os.environ.get("PROBE_GEMM", "")


def _kernel_probe(x, w_mat, scale_x, scale_w):
    m_per, k = x.shape
    _, n_per = w_mat.shape
    xq = x.astype(jnp.float8_e4m3fn)
    wq = w_mat.astype(jnp.float8_e5m2)

    def body(x_ref, w_ref, sx_ref, sw_ref, out_ref):
        scale = sx_ref[0] * sw_ref[0]
        for h in range(N_DEV):
            acc = lax.dot_general(
                x_ref[...], w_ref[...],
                (((1,), (0,)), ((), ())),
                preferred_element_type=jnp.float32,
            )
            y = acc * scale
            out_ref[pl.ds(h * m_per, m_per), :] = y * jax.nn.sigmoid(y)

    return pl.pallas_call(
        body,
        out_shape=jax.ShapeDtypeStruct((N_DEV * m_per, n_per), jnp.float32),
        in_specs=[
            pl.BlockSpec(memory_space=pltpu.VMEM),
            pl.BlockSpec(memory_space=pltpu.VMEM),
            pl.BlockSpec(memory_space=pltpu.SMEM),
            pl.BlockSpec(memory_space=pltpu.SMEM),
        ],
        out_specs=pl.BlockSpec(memory_space=pltpu.VMEM),
        compiler_params=pltpu.CompilerParams(
            vmem_limit_bytes=100 * 1024 * 1024,
        ),
    )(xq, wq, scale_x, scale_w)


def _kernel_real(x, w_mat, scale_x, scale_w):
    m_per, k = x.shape
    k2, n_per = w_mat.shape
    assert k2 == k
    Q = 4
    mq = m_per // Q
    KT = 16
    kt = k // KT

    xq_in = x.astype(jnp.float8_e4m3fn)

    def body(x_ref, w_ref, sx_ref, sw_ref, out_ref,
             wstage, wq, bufL, bufR, bufO, send_sems, recv_sems, wsem):
        my = lax.axis_index("i")
        left = lax.rem(my + (N_DEV - 1), N_DEV)
        right = lax.rem(my + 1, N_DEV)

        barrier_sem = pltpu.get_barrier_semaphore()
        for nbr in (left, right):
            pl.semaphore_signal(
                barrier_sem, inc=1,
                device_id=(nbr,), device_id_type=pl.DeviceIdType.MESH,
            )
        pl.semaphore_wait(barrier_sem, 2)

        scale = sx_ref[0] * sw_ref[0]

        def gemm_store(chunk, row0, rows):
            acc = lax.dot_general(
                chunk, wq[...],
                (((1,), (0,)), ((), ())),
                preferred_element_type=jnp.float32,
            )
            y = acc * scale
            out_ref[pl.ds(row0, rows), :] = y * jax.nn.sigmoid(y)

        def rcopy(src, dst, i, dev):
            return pltpu.make_async_remote_copy(
                src_ref=src, dst_ref=dst,
                send_sem=send_sems.at[i], recv_sem=recv_sems.at[i],
                device_id=(dev,), device_id_type=pl.DeviceIdType.MESH,
            )

        p1R = [rcopy(x_ref.at[pl.ds(q * mq, mq)], bufL.at[pl.ds(q * mq, mq)],
                     q, right) for q in range(Q)]
        p1L = [rcopy(x_ref.at[pl.ds(q * mq, mq)], bufR.at[pl.ds(q * mq, mq)],
                     Q + q, left) for q in range(Q)]
        for r in p1R:
            r.start()
        for r in p1L:
            r.start()
        def wcopy(t, slot):
            return pltpu.make_async_copy(
                w_ref.at[pl.ds(t * kt, kt)], wstage.at[slot], wsem.at[slot],
            )
        wcopy(0, 0).start()
        for t in range(KT):
            if t + 1 < KT:
                wcopy(t + 1, (t + 1) % 2).start()
            wcopy(t, t % 2).wait()
            wq[pl.ds(t * kt, kt), :] = wstage[t % 2].astype(jnp.float8_e5m2)
        gemm_store(x_ref[...], my * m_per, m_per)
        fwd = []
        for q in range(Q):
            p1R[q].wait_recv()
            if q < Q // 2:
                f = rcopy(bufL.at[pl.ds(q * mq, mq)], bufO.at[pl.ds(q * mq, mq)],
                          2 * Q + q, right)
                f.start()
                fwd.append(f)
            gemm_store(bufL[q * mq:(q + 1) * mq, :], left * m_per + q * mq, mq)
            p1L[q].wait_recv()
            if q >= Q // 2:
                f = rcopy(bufR.at[pl.ds(q * mq, mq)], bufO.at[pl.ds(q * mq, mq)],
                          2 * Q + q, left)
                f.start()
                fwd.append(f)
            gemm_store(bufR[q * mq:(q + 1) * mq, :], right * m_per + q * mq, mq)
        opp = lax.rem(my + 2, N_DEV)
        for q in (0, 2, 1, 3):
            fwd[q].wait_recv()
            gemm_store(bufO[q * mq:(q + 1) * mq, :], opp * m_per + q * mq, mq)
        for r in p1R + p1L + fwd:
            r.wait_send()

    return pl.pallas_call(
        body,
        out_shape=jax.ShapeDtypeStruct((N_DEV * m_per, n_per), jnp.float32),
        in_specs=[
            pl.BlockSpec(memory_space=pltpu.VMEM),
            pl.BlockSpec(memory_space=pl.ANY),
            pl.BlockSpec(memory_space=pltpu.SMEM),
            pl.BlockSpec(memory_space=pltpu.SMEM),
        ],
        out_specs=pl.BlockSpec(memory_space=pltpu.VMEM),
        scratch_shapes=[
            pltpu.VMEM((2, kt, n_per), jnp.float32),
            pltpu.VMEM((k, n_per), jnp.float8_e5m2),
            pltpu.VMEM((m_per, k), jnp.float8_e4m3fn),
            pltpu.VMEM((m_per, k), jnp.float8_e4m3fn),
            pltpu.VMEM((m_per, k), jnp.float8_e4m3fn),
            pltpu.SemaphoreType.DMA((3 * Q,)),
            pltpu.SemaphoreType.DMA((3 * Q,)),
            pltpu.SemaphoreType.DMA((2,)),
        ],
        compiler_params=pltpu.CompilerParams(
            collective_id=0,
            vmem_limit_bytes=63 * 1024 * 1024,
        ),
    )(xq_in, w_mat, scale_x, scale_w)


kernel = _kernel_probe if _PROBE else _kernel_real


# device time: 113129 ns/iter; 1.2394x vs baseline; 1.0762x over previous
import os

import jax
import jax.numpy as jnp
from jax import lax
from jax.experimental import pallas as pl
from jax.experimental.pallas import tpu as pltpu

N_DEV = 4

_PROBE = os.environ.get("PROBE_GEMM", "")


def _kernel_probe(x, w_mat, scale_x, scale_w):
    m_per, k = x.shape
    _, n_per = w_mat.shape
    xq = x.astype(jnp.float8_e4m3fn)
    wq = w_mat.astype(jnp.float8_e5m2)

    def body(x_ref, w_ref, sx_ref, sw_ref, out_ref):
        scale = sx_ref[0] * sw_ref[0]
        for h in range(N_DEV):
            acc = lax.dot_general(
                x_ref[...], w_ref[...],
                (((1,), (0,)), ((), ())),
                preferred_element_type=jnp.float32,
            )
            y = acc * scale
            out_ref[pl.ds(h * m_per, m_per), :] = y * jax.nn.sigmoid(y)

    return pl.pallas_call(
        body,
        out_shape=jax.ShapeDtypeStruct((N_DEV * m_per, n_per), jnp.float32),
        in_specs=[
            pl.BlockSpec(memory_space=pltpu.VMEM),
            pl.BlockSpec(memory_space=pltpu.VMEM),
            pl.BlockSpec(memory_space=pltpu.SMEM),
            pl.BlockSpec(memory_space=pltpu.SMEM),
        ],
        out_specs=pl.BlockSpec(memory_space=pltpu.VMEM),
        compiler_params=pltpu.CompilerParams(
            vmem_limit_bytes=100 * 1024 * 1024,
        ),
    )(xq, wq, scale_x, scale_w)


def _kernel_real(x, w_mat, scale_x, scale_w):
    m_per, k = x.shape
    k2, n_per = w_mat.shape
    assert k2 == k
    Q = 4
    mq = m_per // Q
    KT = 16
    kt = k // KT

    xq_in = x.astype(jnp.float8_e4m3fn)

    def body(x_ref, w_ref, sx_ref, sw_ref, out_ref,
             wstage, wq, bufL, bufR, bufO, ystage,
             send_sems, recv_sems, wsem, ysem):
        my = lax.axis_index("i")
        left = lax.rem(my + (N_DEV - 1), N_DEV)
        right = lax.rem(my + 1, N_DEV)

        barrier_sem = pltpu.get_barrier_semaphore()
        for nbr in (left, right):
            pl.semaphore_signal(
                barrier_sem, inc=1,
                device_id=(nbr,), device_id_type=pl.DeviceIdType.MESH,
            )
        pl.semaphore_wait(barrier_sem, 2)

        scale = sx_ref[0] * sw_ref[0]

        pending = [None, None]
        slot_ctr = [0]

        def gemm_store(chunk, row0, rows):
            del rows
            slot = slot_ctr[0] % 2
            slot_ctr[0] += 1
            if pending[slot] is not None:
                pending[slot].wait()
            acc = lax.dot_general(
                chunk, wq[...],
                (((1,), (0,)), ((), ())),
                preferred_element_type=jnp.float32,
            )
            y = acc * scale
            ystage[slot] = y * jax.nn.sigmoid(y)
            cp = pltpu.make_async_copy(
                ystage.at[slot], out_ref.at[pl.ds(row0, mq)], ysem.at[slot],
            )
            cp.start()
            pending[slot] = cp

        def rcopy(src, dst, i, dev):
            return pltpu.make_async_remote_copy(
                src_ref=src, dst_ref=dst,
                send_sem=send_sems.at[i], recv_sem=recv_sems.at[i],
                device_id=(dev,), device_id_type=pl.DeviceIdType.MESH,
            )

        p1R = [rcopy(x_ref.at[pl.ds(q * mq, mq)], bufL.at[pl.ds(q * mq, mq)],
                     q, right) for q in range(Q)]
        p1L = [rcopy(x_ref.at[pl.ds(q * mq, mq)], bufR.at[pl.ds(q * mq, mq)],
                     Q + q, left) for q in range(Q)]
        for r in p1R:
            r.start()
        for r in p1L:
            r.start()
        def wcopy(t, slot):
            return pltpu.make_async_copy(
                w_ref.at[pl.ds(t * kt, kt)], wstage.at[slot], wsem.at[slot],
            )
        wcopy(0, 0).start()
        for t in range(KT):
            if t + 1 < KT:
                wcopy(t + 1, (t + 1) % 2).start()
            wcopy(t, t % 2).wait()
            wq[pl.ds(t * kt, kt), :] = wstage[t % 2].astype(jnp.float8_e5m2)
        for q in range(Q):
            gemm_store(x_ref[q * mq:(q + 1) * mq, :], my * m_per + q * mq, mq)
        fwd = []
        for q in range(Q):
            p1R[q].wait_recv()
            if q < Q // 2:
                f = rcopy(bufL.at[pl.ds(q * mq, mq)], bufO.at[pl.ds(q * mq, mq)],
                          2 * Q + q, right)
                f.start()
                fwd.append(f)
            gemm_store(bufL[q * mq:(q + 1) * mq, :], left * m_per + q * mq, mq)
            p1L[q].wait_recv()
            if q >= Q // 2:
                f = rcopy(bufR.at[pl.ds(q * mq, mq)], bufO.at[pl.ds(q * mq, mq)],
                          2 * Q + q, left)
                f.start()
                fwd.append(f)
            gemm_store(bufR[q * mq:(q + 1) * mq, :], right * m_per + q * mq, mq)
        opp = lax.rem(my + 2, N_DEV)
        for q in (0, 2, 1, 3):
            fwd[q].wait_recv()
            gemm_store(bufO[q * mq:(q + 1) * mq, :], opp * m_per + q * mq, mq)
        for cp in pending:
            if cp is not None:
                cp.wait()
        for r in p1R + p1L + fwd:
            r.wait_send()

    return pl.pallas_call(
        body,
        out_shape=jax.ShapeDtypeStruct((N_DEV * m_per, n_per), jnp.float32),
        in_specs=[
            pl.BlockSpec(memory_space=pltpu.VMEM),
            pl.BlockSpec(memory_space=pl.ANY),
            pl.BlockSpec(memory_space=pltpu.SMEM),
            pl.BlockSpec(memory_space=pltpu.SMEM),
        ],
        out_specs=pl.BlockSpec(memory_space=pl.ANY),
        scratch_shapes=[
            pltpu.VMEM((2, kt, n_per), jnp.float32),
            pltpu.VMEM((k, n_per), jnp.float8_e5m2),
            pltpu.VMEM((m_per, k), jnp.float8_e4m3fn),
            pltpu.VMEM((m_per, k), jnp.float8_e4m3fn),
            pltpu.VMEM((m_per, k), jnp.float8_e4m3fn),
            pltpu.VMEM((2, mq, n_per), jnp.float32),
            pltpu.SemaphoreType.DMA((3 * Q,)),
            pltpu.SemaphoreType.DMA((3 * Q,)),
            pltpu.SemaphoreType.DMA((2,)),
            pltpu.SemaphoreType.DMA((2,)),
        ],
        compiler_params=pltpu.CompilerParams(
            collective_id=0,
            vmem_limit_bytes=63 * 1024 * 1024,
        ),
    )(xq_in, w_mat, scale_x, scale_w)


kernel = _kernel_probe if _PROBE else _kernel_real
